# baseline (speedup 1.0000x reference)
import functools

import jax
import jax.numpy as jnp
from jax import lax
from jax.experimental import pallas as pl
from jax.experimental.pallas import tpu as pltpu
from jax.experimental.pallas import tpu_sc as plsc

_INTERPRET = False


def _select_kernel(z2_ref, pos_ref, q_ref, cb_ref, c2b_ref, out_ref):
    z = q_ref[0]
    cb = cb_ref[...]
    dot = jax.lax.dot_general(cb, z, (((1,), (0,)), ((), ())),
                              preferred_element_type=jnp.float32)
    z2 = z2_ref[0, 0]
    c2 = c2b_ref[:, 0:1]
    d2 = (z2 + c2) - 2.0 * dot

    u = jax.lax.bitcast_convert_type(d2, jnp.int32)
    key = u ^ ((u >> 31) & jnp.int32(0x7FFFFFFF))

    kk = key.shape[0]
    tn = key.shape[1]
    t = pos_ref[0, 0].astype(jnp.float32)
    iota = jax.lax.broadcasted_iota(jnp.int32, key.shape, 0)

    minkey = jnp.min(key)
    maxkey = jnp.max(key)
    dxor = minkey ^ maxkey
    dxf = jax.lax.bitcast_convert_type(dxor.astype(jnp.float32), jnp.int32)
    nb = jnp.where(dxor == 0, jnp.int32(0), (dxf >> 23) - 127)
    nb = jnp.where(dxor < 0, jnp.int32(31), nb)

    def _init_prefix(_):
        p0 = minkey & ~((jnp.int32(1) << (nb + 1)) - 1)
        return (jnp.minimum(nb, 30),
                jnp.broadcast_to(p0[None, None], (1, tn)),
                jnp.zeros((1, tn), jnp.float32),
                jnp.full((1, tn), jnp.float32(kk)))

    def _init_sign(_):
        cnt0 = jnp.sum((key < 0).astype(jnp.float32), axis=0, keepdims=True)
        neg = cnt0 > t
        return (jnp.int32(30),
                jnp.where(neg, jnp.int32(-(2 ** 31)), jnp.int32(0)),
                jnp.where(neg, jnp.float32(0.0), cnt0),
                jnp.where(neg, cnt0, jnp.float32(kk)))

    b0, p_0, lo0, hi0 = jax.lax.cond(nb >= 31, _init_sign, _init_prefix, 0)

    def _cond(carry):
        b, _, _, _, maxw = carry
        return jnp.logical_and(b >= 0, maxw > 1.5)

    def _body(carry):
        b, p, lo, hi, _ = carry
        c = p + (jnp.int32(1) << b)
        cnt = jnp.sum((key < c).astype(jnp.float32), axis=0, keepdims=True)
        acc = cnt <= t
        p = jnp.where(acc, c, p)
        lo = jnp.where(acc, cnt, lo)
        hi = jnp.where(acc, hi, cnt)
        return (b - 1, p, lo, hi, jnp.max(hi - lo))

    b, P, lo, hi, maxw = jax.lax.while_loop(
        _cond, _body, (b0, p_0, lo0, hi0, jnp.float32(kk)))

    ub = P + ((jnp.int32(1) << (b + 1)) - 1)
    win = (key >= P) & (key <= ub)

    def _unique_path(_):
        return jnp.sum(jnp.where(win, iota, 0), axis=0, keepdims=True)

    def _tie_path(_):
        t2 = t - lo

        def _first(_):
            return jnp.min(jnp.where(win, iota, jnp.int32(kk)),
                           axis=0, keepdims=True)

        def _search(_):
            q = jnp.zeros(t.shape, jnp.int32)
            for bb in range(9, -1, -1):
                cc = q + jnp.int32(1 << bb)
                cnt = jnp.sum(jnp.where(win & (iota < cc), 1.0, 0.0),
                              axis=0, keepdims=True)
                q = jnp.where(cnt <= t2, cc, q)
            return q

        return jax.lax.cond(jnp.max(t2) <= 0.5, _first, _search, 0)

    idx = jax.lax.cond(maxw <= 1.5, _unique_path, _tie_path, 0)
    out_ref[0, 0] = idx


def _run_select(q, cbk, c2, z2, pos, tn):
    bsz, d, h, w = q.shape
    hw = h * w
    k = cbk.shape[0]
    g = hw // tn
    qr = q.reshape(bsz, d, hw)
    z2r = z2.reshape(bsz, g, 1, tn)
    posr = pos.reshape(bsz, g, 1, tn)
    c2b = jnp.broadcast_to(c2[:, None], (k, 128))

    idx = pl.pallas_call(
        _select_kernel,
        grid=(bsz, g),
        in_specs=[
            pl.BlockSpec((1, 1, 1, tn), lambda b, j: (b, j, 0, 0)),
            pl.BlockSpec((1, 1, 1, tn), lambda b, j: (b, j, 0, 0)),
            pl.BlockSpec((1, d, tn), lambda b, j: (b, 0, j)),
            pl.BlockSpec((k, d), lambda b, j: (0, 0)),
            pl.BlockSpec((k, 128), lambda b, j: (0, 0)),
        ],
        out_specs=pl.BlockSpec((1, 1, 1, tn), lambda b, j: (b, j, 0, 0)),
        out_shape=jax.ShapeDtypeStruct((bsz, g, 1, tn), jnp.int32),
        compiler_params=pltpu.CompilerParams(
            dimension_semantics=("parallel", "arbitrary")),
        interpret=_INTERPRET,
    )(z2r, posr, qr, cbk, c2b)
    return idx.reshape(bsz * hw)


def _sc_gather(cbk, idx_flat):
    n_total = idx_flat.shape[0]
    d = cbk.shape[1]
    info = plsc.get_sparse_core_info()
    nw = info.num_cores * info.num_subcores
    b_per_w = n_total // nw
    ch = min(b_per_w, 256)
    nch = b_per_w // ch
    mesh = plsc.VectorSubcoreMesh(core_axis_name="c", subcore_axis_name="s")

    @functools.partial(
        pl.kernel, mesh=mesh,
        out_type=jax.ShapeDtypeStruct((n_total, d), jnp.float32),
        scratch_types=[
            pltpu.VMEM((ch,), jnp.int32),
            pltpu.VMEM((ch, d), jnp.float32),
            pltpu.SemaphoreType.DMA,
        ],
    )
    def k(cb_hbm, idx_hbm, out_hbm, idx_v, rows_v, sem):
        wid = lax.axis_index("s") * info.num_cores + lax.axis_index("c")
        base = wid * b_per_w
        for c in range(nch):
            off = base + c * ch
            pltpu.sync_copy(idx_hbm.at[pl.ds(off, ch)], idx_v)
            pltpu.async_copy(cb_hbm.at[idx_v], rows_v, sem).wait()
            pltpu.sync_copy(rows_v, out_hbm.at[pl.ds(off, ch)])

    return k(cbk, idx_flat)


def _blend_kernel(s_ref, q_ref, m_ref, o_ref):
    st = jnp.transpose(s_ref[0, 0], (1, 0))
    m = m_ref[0, 0]
    o_ref[0] = m * st + (1.0 - m) * q_ref[0]


def _run_blend(sampled_nd, q, mask2d, tnb):
    bsz, d, h, w = q.shape
    hw = h * w
    g = hw // tnb
    qr = q.reshape(bsz, d, hw)
    sr = sampled_nd.reshape(bsz, g, tnb, d)
    mr = mask2d.reshape(bsz, g, 1, tnb)
    out = pl.pallas_call(
        _blend_kernel,
        grid=(bsz, g),
        in_specs=[
            pl.BlockSpec((1, 1, tnb, d), lambda b, j: (b, j, 0, 0)),
            pl.BlockSpec((1, d, tnb), lambda b, j: (b, 0, j)),
            pl.BlockSpec((1, 1, 1, tnb), lambda b, j: (b, j, 0, 0)),
        ],
        out_specs=pl.BlockSpec((1, d, tnb), lambda b, j: (b, 0, j)),
        out_shape=jax.ShapeDtypeStruct((bsz, d, hw), jnp.float32),
        compiler_params=pltpu.CompilerParams(
            dimension_semantics=("parallel", "arbitrary")),
        interpret=_INTERPRET,
    )(sr, qr, mr)
    return out.reshape(bsz, d, h, w)


def _level_prep(q, cbk, strength, u):
    bsz, d, h, w = q.shape
    hw = h * w
    k = cbk.shape[0]
    flat = jnp.transpose(q, (0, 2, 3, 1)).reshape(bsz, hw, d)
    z2 = jnp.sum(flat ** 2, axis=-1)
    c2 = jnp.sum(cbk ** 2, axis=-1)
    skip = int(0.05 * k)
    s = strength.reshape(())
    n = jnp.maximum(jnp.floor(s * (k - skip)).astype(jnp.int32), 1)
    offs = jnp.floor(u * n.astype(jnp.float32)).astype(jnp.int32)
    pos = jnp.minimum(skip + offs, k - 1)
    return flat, z2, c2, pos


def _level(q, cbk, strength, u, mask2d):
    flat, z2, c2, pos = _level_prep(q, cbk, strength, u)
    idx = _run_select(q, cbk, c2, z2, pos, min(2048, q.shape[2] * q.shape[3]))
    sampled_nd = _sc_gather(cbk, idx)
    return _run_blend(sampled_nd, q, mask2d, 512)


def kernel(q_fine, q_coarse, M, cb_fine, cb_coarse, strength_fine, strength_coarse):
    M_float = M.astype(jnp.float32)
    fh, fw = q_fine.shape[-2], q_fine.shape[-1]
    ch, cw = q_coarse.shape[-2], q_coarse.shape[-1]
    mh, mw = M.shape[-2], M.shape[-1]
    m_fine = M_float[:, 0, :: mh // fh, :: mw // fw].reshape(M.shape[0], fh * fw)
    m_coarse = M_float[:, 0, :: mh // ch, :: mw // cw].reshape(M.shape[0], ch * cw)

    key = jax.random.key(42)
    kc, kf = jax.random.split(key)
    bsz = q_fine.shape[0]
    u_c = jax.random.uniform(kc, (bsz, ch * cw))
    u_f = jax.random.uniform(kf, (bsz, fh * fw))

    out_c = _level(q_coarse, cb_coarse, strength_coarse, u_c, m_coarse)
    out_f = _level(q_fine, cb_fine, strength_fine, u_f, m_fine)
    return (out_f, out_c)


# speedup vs baseline: 1.0228x; 1.0228x over previous
import functools

import jax
import jax.numpy as jnp
from jax import lax
from jax.experimental import pallas as pl
from jax.experimental.pallas import tpu as pltpu
from jax.experimental.pallas import tpu_sc as plsc

_INTERPRET = False


def _select_kernel(z2_ref, pos_ref, q_ref, cb_ref, c2b_ref, out_ref):
    z = q_ref[0]
    cb = cb_ref[...]
    dot = jax.lax.dot_general(cb, z, (((1,), (0,)), ((), ())),
                              preferred_element_type=jnp.float32)
    z2 = z2_ref[0, 0]
    c2 = c2b_ref[:, 0:1]
    d2 = (z2 + c2) - 2.0 * dot

    u = jax.lax.bitcast_convert_type(d2, jnp.int32)
    key = u ^ ((u >> 31) & jnp.int32(0x7FFFFFFF))

    kk = key.shape[0]
    tn = key.shape[1]
    t = pos_ref[0, 0].astype(jnp.float32)
    iota = jax.lax.broadcasted_iota(jnp.int32, key.shape, 0)

    minkey = jnp.min(key)
    maxkey = jnp.max(key)
    dxor = minkey ^ maxkey
    dxf = jax.lax.bitcast_convert_type(dxor.astype(jnp.float32), jnp.int32)
    nb = jnp.where(dxor == 0, jnp.int32(0), (dxf >> 23) - 127)
    nb = jnp.where(dxor < 0, jnp.int32(31), nb)

    def _init_prefix(_):
        p0 = minkey & ~((jnp.int32(1) << (nb + 1)) - 1)
        return (jnp.minimum(nb, 30),
                jnp.broadcast_to(p0[None, None], (1, tn)),
                jnp.zeros((1, tn), jnp.float32),
                jnp.full((1, tn), jnp.float32(kk)))

    def _init_sign(_):
        cnt0 = jnp.sum((key < 0).astype(jnp.float32), axis=0, keepdims=True)
        neg = cnt0 > t
        return (jnp.int32(30),
                jnp.where(neg, jnp.int32(-(2 ** 31)), jnp.int32(0)),
                jnp.where(neg, jnp.float32(0.0), cnt0),
                jnp.where(neg, cnt0, jnp.float32(kk)))

    b0, p_0, lo0, hi0 = jax.lax.cond(nb >= 31, _init_sign, _init_prefix, 0)

    def _cond(carry):
        b, _, _, _, maxw = carry
        return jnp.logical_and(b >= 0, maxw > 1.5)

    def _body(carry):
        b, p, lo, hi, _ = carry
        c = p + (jnp.int32(1) << b)
        cnt = jnp.sum((key < c).astype(jnp.float32), axis=0, keepdims=True)
        acc = cnt <= t
        p = jnp.where(acc, c, p)
        lo = jnp.where(acc, cnt, lo)
        hi = jnp.where(acc, hi, cnt)
        return (b - 1, p, lo, hi, jnp.max(hi - lo))

    b, P, lo, hi, maxw = jax.lax.while_loop(
        _cond, _body, (b0, p_0, lo0, hi0, jnp.float32(kk)))

    ub = P + ((jnp.int32(1) << (b + 1)) - 1)
    win = (key >= P) & (key <= ub)

    def _unique_path(_):
        return jnp.sum(jnp.where(win, iota, 0), axis=0, keepdims=True)

    def _tie_path(_):
        t2 = t - lo

        def _first(_):
            return jnp.min(jnp.where(win, iota, jnp.int32(kk)),
                           axis=0, keepdims=True)

        def _search(_):
            q = jnp.zeros(t.shape, jnp.int32)
            for bb in range(9, -1, -1):
                cc = q + jnp.int32(1 << bb)
                cnt = jnp.sum(jnp.where(win & (iota < cc), 1.0, 0.0),
                              axis=0, keepdims=True)
                q = jnp.where(cnt <= t2, cc, q)
            return q

        return jax.lax.cond(jnp.max(t2) <= 0.5, _first, _search, 0)

    idx = jax.lax.cond(maxw <= 1.5, _unique_path, _tie_path, 0)
    out_ref[0, 0] = idx


def _run_select(q, cbk, c2, z2, pos, tn):
    bsz, d, h, w = q.shape
    hw = h * w
    k = cbk.shape[0]
    g = hw // tn
    qr = q.reshape(bsz, d, hw)
    z2r = z2.reshape(bsz, g, 1, tn)
    posr = pos.reshape(bsz, g, 1, tn)
    c2b = jnp.broadcast_to(c2[:, None], (k, 128))

    idx = pl.pallas_call(
        _select_kernel,
        grid=(bsz, g),
        in_specs=[
            pl.BlockSpec((1, 1, 1, tn), lambda b, j: (b, j, 0, 0)),
            pl.BlockSpec((1, 1, 1, tn), lambda b, j: (b, j, 0, 0)),
            pl.BlockSpec((1, d, tn), lambda b, j: (b, 0, j)),
            pl.BlockSpec((k, d), lambda b, j: (0, 0)),
            pl.BlockSpec((k, 128), lambda b, j: (0, 0)),
        ],
        out_specs=pl.BlockSpec((1, 1, 1, tn), lambda b, j: (b, j, 0, 0)),
        out_shape=jax.ShapeDtypeStruct((bsz, g, 1, tn), jnp.int32),
        compiler_params=pltpu.CompilerParams(
            dimension_semantics=("parallel", "arbitrary")),
        interpret=_INTERPRET,
    )(z2r, posr, qr, cbk, c2b)
    return idx.reshape(bsz * hw)


def _sc_gather(cbk, idx_flat):
    n_total = idx_flat.shape[0]
    d = cbk.shape[1]
    info = plsc.get_sparse_core_info()
    nw = info.num_cores * info.num_subcores
    b_per_w = n_total // nw
    ch = min(b_per_w, 256)
    nch = b_per_w // ch
    mesh = plsc.VectorSubcoreMesh(core_axis_name="c", subcore_axis_name="s")

    @functools.partial(
        pl.kernel, mesh=mesh,
        out_type=jax.ShapeDtypeStruct((n_total, d), jnp.float32),
        scratch_types=[
            pltpu.VMEM((ch,), jnp.int32),
            pltpu.VMEM((ch, d), jnp.float32),
            pltpu.SemaphoreType.DMA,
        ],
    )
    def k(cb_hbm, idx_hbm, out_hbm, idx_v, rows_v, sem):
        wid = lax.axis_index("s") * info.num_cores + lax.axis_index("c")
        base = wid * b_per_w
        for c in range(nch):
            off = base + c * ch
            pltpu.sync_copy(idx_hbm.at[pl.ds(off, ch)], idx_v)
            pltpu.async_copy(cb_hbm.at[idx_v], rows_v, sem).wait()
            pltpu.sync_copy(rows_v, out_hbm.at[pl.ds(off, ch)])

    return k(cbk, idx_flat)


def _blend_kernel(s_ref, q_ref, m_ref, o_ref):
    st = jnp.transpose(s_ref[0, 0], (1, 0))
    m = m_ref[0, 0]
    o_ref[0] = m * st + (1.0 - m) * q_ref[0]


def _run_blend(sampled_nd, q, mask2d, tnb):
    bsz, d, h, w = q.shape
    hw = h * w
    g = hw // tnb
    qr = q.reshape(bsz, d, hw)
    sr = sampled_nd.reshape(bsz, g, tnb, d)
    mr = mask2d.reshape(bsz, g, 1, tnb)
    out = pl.pallas_call(
        _blend_kernel,
        grid=(bsz, g),
        in_specs=[
            pl.BlockSpec((1, 1, tnb, d), lambda b, j: (b, j, 0, 0)),
            pl.BlockSpec((1, d, tnb), lambda b, j: (b, 0, j)),
            pl.BlockSpec((1, 1, 1, tnb), lambda b, j: (b, j, 0, 0)),
        ],
        out_specs=pl.BlockSpec((1, d, tnb), lambda b, j: (b, 0, j)),
        out_shape=jax.ShapeDtypeStruct((bsz, d, hw), jnp.float32),
        compiler_params=pltpu.CompilerParams(
            dimension_semantics=("parallel", "arbitrary")),
        interpret=_INTERPRET,
    )(sr, qr, mr)
    return out.reshape(bsz, d, h, w)


def _level_prep(q, cbk, strength, u):
    bsz, d, h, w = q.shape
    hw = h * w
    k = cbk.shape[0]
    flat = jnp.transpose(q, (0, 2, 3, 1)).reshape(bsz, hw, d)
    z2 = jnp.sum(flat ** 2, axis=-1)
    c2 = jnp.sum(cbk ** 2, axis=-1)
    skip = int(0.05 * k)
    s = strength.reshape(())
    n = jnp.maximum(jnp.floor(s * (k - skip)).astype(jnp.int32), 1)
    offs = jnp.floor(u * n.astype(jnp.float32)).astype(jnp.int32)
    pos = jnp.minimum(skip + offs, k - 1)
    return flat, z2, c2, pos


def _level(q, cbk, strength, u, mask2d):
    flat, z2, c2, pos = _level_prep(q, cbk, strength, u)
    idx = _run_select(q, cbk, c2, z2, pos, min(2048, q.shape[2] * q.shape[3]))
    sampled_nd = _sc_gather(cbk, idx)
    return _run_blend(sampled_nd, q, mask2d, 1024)


def kernel(q_fine, q_coarse, M, cb_fine, cb_coarse, strength_fine, strength_coarse):
    M_float = M.astype(jnp.float32)
    fh, fw = q_fine.shape[-2], q_fine.shape[-1]
    ch, cw = q_coarse.shape[-2], q_coarse.shape[-1]
    mh, mw = M.shape[-2], M.shape[-1]
    m_fine = M_float[:, 0, :: mh // fh, :: mw // fw].reshape(M.shape[0], fh * fw)
    m_coarse = M_float[:, 0, :: mh // ch, :: mw // cw].reshape(M.shape[0], ch * cw)

    key = jax.random.key(42)
    kc, kf = jax.random.split(key)
    bsz = q_fine.shape[0]
    u_c = jax.random.uniform(kc, (bsz, ch * cw))
    u_f = jax.random.uniform(kf, (bsz, fh * fw))

    out_c = _level(q_coarse, cb_coarse, strength_coarse, u_c, m_coarse)
    out_f = _level(q_fine, cb_fine, strength_fine, u_f, m_fine)
    return (out_f, out_c)


# speedup vs baseline: 1.0347x; 1.0116x over previous
import functools

import jax
import jax.numpy as jnp
from jax import lax
from jax.experimental import pallas as pl
from jax.experimental.pallas import tpu as pltpu
from jax.experimental.pallas import tpu_sc as plsc

_INTERPRET = False


def _select_kernel(z2_ref, pos_ref, q_ref, cb_ref, c2b_ref, out_ref):
    z = q_ref[0]
    cb = cb_ref[...]
    dot = jax.lax.dot_general(cb, z, (((1,), (0,)), ((), ())),
                              preferred_element_type=jnp.float32)
    z2 = z2_ref[0, 0]
    c2 = c2b_ref[:, 0:1]
    d2 = (z2 + c2) - 2.0 * dot

    u = jax.lax.bitcast_convert_type(d2, jnp.int32)
    key = u ^ ((u >> 31) & jnp.int32(0x7FFFFFFF))

    kk = key.shape[0]
    tn = key.shape[1]
    t = pos_ref[0, 0].astype(jnp.float32)
    iota = jax.lax.broadcasted_iota(jnp.int32, key.shape, 0)

    minkey = jnp.min(key)
    maxkey = jnp.max(key)
    dxor = minkey ^ maxkey
    dxf = jax.lax.bitcast_convert_type(dxor.astype(jnp.float32), jnp.int32)
    nb = jnp.where(dxor == 0, jnp.int32(0), (dxf >> 23) - 127)
    nb = jnp.where(dxor < 0, jnp.int32(31), nb)

    def _init_prefix(_):
        p0 = minkey & ~((jnp.int32(1) << (nb + 1)) - 1)
        return (jnp.minimum(nb, 30),
                jnp.broadcast_to(p0[None, None], (1, tn)),
                jnp.zeros((1, tn), jnp.float32),
                jnp.full((1, tn), jnp.float32(kk)))

    def _init_sign(_):
        cnt0 = jnp.sum((key < 0).astype(jnp.float32), axis=0, keepdims=True)
        neg = cnt0 > t
        return (jnp.int32(30),
                jnp.where(neg, jnp.int32(-(2 ** 31)), jnp.int32(0)),
                jnp.where(neg, jnp.float32(0.0), cnt0),
                jnp.where(neg, cnt0, jnp.float32(kk)))

    b0, p_0, lo0, hi0 = jax.lax.cond(nb >= 31, _init_sign, _init_prefix, 0)

    def _cond(carry):
        b, _, _, _, maxw = carry
        return jnp.logical_and(b >= 0, maxw > 1.5)

    def _body(carry):
        b, p, lo, hi, _ = carry
        c = p + (jnp.int32(1) << b)
        cnt = jnp.sum((key < c).astype(jnp.float32), axis=0, keepdims=True)
        acc = cnt <= t
        p = jnp.where(acc, c, p)
        lo = jnp.where(acc, cnt, lo)
        hi = jnp.where(acc, hi, cnt)
        return (b - 1, p, lo, hi, jnp.max(hi - lo))

    b, P, lo, hi, maxw = jax.lax.while_loop(
        _cond, _body, (b0, p_0, lo0, hi0, jnp.float32(kk)))

    ub = P + ((jnp.int32(1) << (b + 1)) - 1)
    win = (key >= P) & (key <= ub)

    def _unique_path(_):
        return jnp.sum(jnp.where(win, iota, 0), axis=0, keepdims=True)

    def _tie_path(_):
        t2 = t - lo

        def _first(_):
            return jnp.min(jnp.where(win, iota, jnp.int32(kk)),
                           axis=0, keepdims=True)

        def _search(_):
            q = jnp.zeros(t.shape, jnp.int32)
            for bb in range(9, -1, -1):
                cc = q + jnp.int32(1 << bb)
                cnt = jnp.sum(jnp.where(win & (iota < cc), 1.0, 0.0),
                              axis=0, keepdims=True)
                q = jnp.where(cnt <= t2, cc, q)
            return q

        return jax.lax.cond(jnp.max(t2) <= 0.5, _first, _search, 0)

    idx = jax.lax.cond(maxw <= 1.5, _unique_path, _tie_path, 0)
    out_ref[0, 0] = idx


def _run_select(q, cbk, c2, z2, pos, tn):
    bsz, d, h, w = q.shape
    hw = h * w
    k = cbk.shape[0]
    g = hw // tn
    qr = q.reshape(bsz, d, hw)
    z2r = z2.reshape(bsz, g, 1, tn)
    posr = pos.reshape(bsz, g, 1, tn)
    c2b = jnp.broadcast_to(c2[:, None], (k, 128))

    idx = pl.pallas_call(
        _select_kernel,
        grid=(bsz, g),
        in_specs=[
            pl.BlockSpec((1, 1, 1, tn), lambda b, j: (b, j, 0, 0)),
            pl.BlockSpec((1, 1, 1, tn), lambda b, j: (b, j, 0, 0)),
            pl.BlockSpec((1, d, tn), lambda b, j: (b, 0, j)),
            pl.BlockSpec((k, d), lambda b, j: (0, 0)),
            pl.BlockSpec((k, 128), lambda b, j: (0, 0)),
        ],
        out_specs=pl.BlockSpec((1, 1, 1, tn), lambda b, j: (b, j, 0, 0)),
        out_shape=jax.ShapeDtypeStruct((bsz, g, 1, tn), jnp.int32),
        compiler_params=pltpu.CompilerParams(
            dimension_semantics=("parallel", "arbitrary")),
        interpret=_INTERPRET,
    )(z2r, posr, qr, cbk, c2b)
    return idx.reshape(bsz * hw)


def _sc_gather(cbk, idx_flat):
    n_total = idx_flat.shape[0]
    d = cbk.shape[1]
    info = plsc.get_sparse_core_info()
    nw = info.num_cores * info.num_subcores
    b_per_w = n_total // nw
    ch = min(b_per_w, 256)
    nch = b_per_w // ch
    mesh = plsc.VectorSubcoreMesh(core_axis_name="c", subcore_axis_name="s")

    @functools.partial(
        pl.kernel, mesh=mesh,
        out_type=jax.ShapeDtypeStruct((n_total, d), jnp.float32),
        scratch_types=[
            pltpu.VMEM((ch,), jnp.int32),
            pltpu.VMEM((ch, d), jnp.float32),
            pltpu.SemaphoreType.DMA,
        ],
    )
    def k(cb_hbm, idx_hbm, out_hbm, idx_v, rows_v, sem):
        wid = lax.axis_index("s") * info.num_cores + lax.axis_index("c")
        base = wid * b_per_w
        for c in range(nch):
            off = base + c * ch
            pltpu.sync_copy(idx_hbm.at[pl.ds(off, ch)], idx_v)
            pltpu.async_copy(cb_hbm.at[idx_v], rows_v, sem).wait()
            pltpu.sync_copy(rows_v, out_hbm.at[pl.ds(off, ch)])

    return k(cbk, idx_flat)


def _blend_kernel(s_ref, q_ref, m_ref, o_ref):
    st = jnp.transpose(s_ref[0, 0], (1, 0))
    m = m_ref[0, 0]
    o_ref[0] = m * st + (1.0 - m) * q_ref[0]


def _run_blend(sampled_nd, q, mask2d, tnb):
    bsz, d, h, w = q.shape
    hw = h * w
    g = hw // tnb
    qr = q.reshape(bsz, d, hw)
    sr = sampled_nd.reshape(bsz, g, tnb, d)
    mr = mask2d.reshape(bsz, g, 1, tnb)
    out = pl.pallas_call(
        _blend_kernel,
        grid=(bsz, g),
        in_specs=[
            pl.BlockSpec((1, 1, tnb, d), lambda b, j: (b, j, 0, 0)),
            pl.BlockSpec((1, d, tnb), lambda b, j: (b, 0, j)),
            pl.BlockSpec((1, 1, 1, tnb), lambda b, j: (b, j, 0, 0)),
        ],
        out_specs=pl.BlockSpec((1, d, tnb), lambda b, j: (b, 0, j)),
        out_shape=jax.ShapeDtypeStruct((bsz, d, hw), jnp.float32),
        compiler_params=pltpu.CompilerParams(
            dimension_semantics=("parallel", "arbitrary")),
        interpret=_INTERPRET,
    )(sr, qr, mr)
    return out.reshape(bsz, d, h, w)


def _level_prep(q, cbk, strength, u):
    bsz, d, h, w = q.shape
    hw = h * w
    k = cbk.shape[0]
    flat = jnp.transpose(q, (0, 2, 3, 1)).reshape(bsz, hw, d)
    z2 = jnp.sum(flat ** 2, axis=-1)
    c2 = jnp.sum(cbk ** 2, axis=-1)
    skip = int(0.05 * k)
    s = strength.reshape(())
    n = jnp.maximum(jnp.floor(s * (k - skip)).astype(jnp.int32), 1)
    offs = jnp.floor(u * n.astype(jnp.float32)).astype(jnp.int32)
    pos = jnp.minimum(skip + offs, k - 1)
    return flat, z2, c2, pos


def _level(q, cbk, strength, u, mask2d):
    flat, z2, c2, pos = _level_prep(q, cbk, strength, u)
    idx = _run_select(q, cbk, c2, z2, pos, min(2048, q.shape[2] * q.shape[3]))
    sampled_nd = _sc_gather(cbk, idx)
    return _run_blend(sampled_nd, q, mask2d, 2048)


def kernel(q_fine, q_coarse, M, cb_fine, cb_coarse, strength_fine, strength_coarse):
    M_float = M.astype(jnp.float32)
    fh, fw = q_fine.shape[-2], q_fine.shape[-1]
    ch, cw = q_coarse.shape[-2], q_coarse.shape[-1]
    mh, mw = M.shape[-2], M.shape[-1]
    m_fine = M_float[:, 0, :: mh // fh, :: mw // fw].reshape(M.shape[0], fh * fw)
    m_coarse = M_float[:, 0, :: mh // ch, :: mw // cw].reshape(M.shape[0], ch * cw)

    key = jax.random.key(42)
    kc, kf = jax.random.split(key)
    bsz = q_fine.shape[0]
    u_c = jax.random.uniform(kc, (bsz, ch * cw))
    u_f = jax.random.uniform(kf, (bsz, fh * fw))

    out_c = _level(q_coarse, cb_coarse, strength_coarse, u_c, m_coarse)
    out_f = _level(q_fine, cb_fine, strength_fine, u_f, m_fine)
    return (out_f, out_c)


# speedup vs baseline: 1.0400x; 1.0052x over previous
import functools

import jax
import jax.numpy as jnp
from jax import lax
from jax.experimental import pallas as pl
from jax.experimental.pallas import tpu as pltpu
from jax.experimental.pallas import tpu_sc as plsc

_INTERPRET = False


def _select_kernel(z2_ref, pos_ref, q_ref, cb_ref, c2b_ref, out_ref):
    z = q_ref[0]
    cb = cb_ref[...]
    dot = jax.lax.dot_general(cb, z, (((1,), (0,)), ((), ())),
                              preferred_element_type=jnp.float32)
    z2 = z2_ref[0, 0]
    c2 = c2b_ref[:, 0:1]
    d2 = (z2 + c2) - 2.0 * dot

    u = jax.lax.bitcast_convert_type(d2, jnp.int32)
    key = u ^ ((u >> 31) & jnp.int32(0x7FFFFFFF))

    kk = key.shape[0]
    tn = key.shape[1]
    t = pos_ref[0, 0].astype(jnp.float32)
    iota = jax.lax.broadcasted_iota(jnp.int32, key.shape, 0)

    minkey = jnp.min(key)
    maxkey = jnp.max(key)
    dxor = minkey ^ maxkey
    dxf = jax.lax.bitcast_convert_type(dxor.astype(jnp.float32), jnp.int32)
    nb = jnp.where(dxor == 0, jnp.int32(0), (dxf >> 23) - 127)
    nb = jnp.where(dxor < 0, jnp.int32(31), nb)

    def _init_prefix(_):
        p0 = minkey & ~((jnp.int32(1) << (nb + 1)) - 1)
        return (jnp.minimum(nb, 30),
                jnp.broadcast_to(p0[None, None], (1, tn)),
                jnp.zeros((1, tn), jnp.float32),
                jnp.full((1, tn), jnp.float32(kk)))

    def _init_sign(_):
        cnt0 = jnp.sum((key < 0).astype(jnp.float32), axis=0, keepdims=True)
        neg = cnt0 > t
        return (jnp.int32(30),
                jnp.where(neg, jnp.int32(-(2 ** 31)), jnp.int32(0)),
                jnp.where(neg, jnp.float32(0.0), cnt0),
                jnp.where(neg, cnt0, jnp.float32(kk)))

    b0, p_0, lo0, hi0 = jax.lax.cond(nb >= 31, _init_sign, _init_prefix, 0)

    def _cond(carry):
        b, _, _, _, maxw = carry
        return jnp.logical_and(b >= 0, maxw > 1.5)

    def _body(carry):
        b, p, lo, hi, _ = carry
        c = p + (jnp.int32(1) << b)
        cnt = jnp.sum((key < c).astype(jnp.float32), axis=0, keepdims=True)
        acc = cnt <= t
        p = jnp.where(acc, c, p)
        lo = jnp.where(acc, cnt, lo)
        hi = jnp.where(acc, hi, cnt)
        return (b - 1, p, lo, hi, jnp.max(hi - lo))

    b, P, lo, hi, maxw = jax.lax.while_loop(
        _cond, _body, (b0, p_0, lo0, hi0, jnp.float32(kk)))

    ub = P + ((jnp.int32(1) << (b + 1)) - 1)
    win = (key >= P) & (key <= ub)

    def _unique_path(_):
        return jnp.sum(jnp.where(win, iota, 0), axis=0, keepdims=True)

    def _tie_path(_):
        t2 = t - lo

        def _first(_):
            return jnp.min(jnp.where(win, iota, jnp.int32(kk)),
                           axis=0, keepdims=True)

        def _search(_):
            q = jnp.zeros(t.shape, jnp.int32)
            for bb in range(9, -1, -1):
                cc = q + jnp.int32(1 << bb)
                cnt = jnp.sum(jnp.where(win & (iota < cc), 1.0, 0.0),
                              axis=0, keepdims=True)
                q = jnp.where(cnt <= t2, cc, q)
            return q

        return jax.lax.cond(jnp.max(t2) <= 0.5, _first, _search, 0)

    idx = jax.lax.cond(maxw <= 1.5, _unique_path, _tie_path, 0)
    out_ref[0, 0] = idx


def _run_select(q, cbk, c2, z2, pos, tn):
    bsz, d, h, w = q.shape
    hw = h * w
    k = cbk.shape[0]
    g = hw // tn
    qr = q.reshape(bsz, d, hw)
    z2r = z2.reshape(bsz, g, 1, tn)
    posr = pos.reshape(bsz, g, 1, tn)
    c2b = jnp.broadcast_to(c2[:, None], (k, 128))

    idx = pl.pallas_call(
        _select_kernel,
        grid=(bsz, g),
        in_specs=[
            pl.BlockSpec((1, 1, 1, tn), lambda b, j: (b, j, 0, 0)),
            pl.BlockSpec((1, 1, 1, tn), lambda b, j: (b, j, 0, 0)),
            pl.BlockSpec((1, d, tn), lambda b, j: (b, 0, j)),
            pl.BlockSpec((k, d), lambda b, j: (0, 0)),
            pl.BlockSpec((k, 128), lambda b, j: (0, 0)),
        ],
        out_specs=pl.BlockSpec((1, 1, 1, tn), lambda b, j: (b, j, 0, 0)),
        out_shape=jax.ShapeDtypeStruct((bsz, g, 1, tn), jnp.int32),
        compiler_params=pltpu.CompilerParams(
            dimension_semantics=("parallel", "arbitrary")),
        interpret=_INTERPRET,
    )(z2r, posr, qr, cbk, c2b)
    return idx.reshape(bsz * hw)


def _sc_gather(cbk, idx_flat):
    n_total = idx_flat.shape[0]
    d = cbk.shape[1]
    info = plsc.get_sparse_core_info()
    nw = info.num_cores * info.num_subcores
    b_per_w = n_total // nw
    ch = min(b_per_w, 256)
    nch = b_per_w // ch
    mesh = plsc.VectorSubcoreMesh(core_axis_name="c", subcore_axis_name="s")

    @functools.partial(
        pl.kernel, mesh=mesh,
        out_type=jax.ShapeDtypeStruct((n_total, d), jnp.float32),
        scratch_types=[
            pltpu.VMEM((ch,), jnp.int32),
            pltpu.VMEM((ch, d), jnp.float32),
            pltpu.SemaphoreType.DMA,
        ],
    )
    def k(cb_hbm, idx_hbm, out_hbm, idx_v, rows_v, sem):
        wid = lax.axis_index("s") * info.num_cores + lax.axis_index("c")
        base = wid * b_per_w
        for c in range(nch):
            off = base + c * ch
            pltpu.sync_copy(idx_hbm.at[pl.ds(off, ch)], idx_v)
            pltpu.async_copy(cb_hbm.at[idx_v], rows_v, sem).wait()
            pltpu.sync_copy(rows_v, out_hbm.at[pl.ds(off, ch)])

    return k(cbk, idx_flat)


def _blend_kernel(s_ref, q_ref, m_ref, o_ref):
    st = jnp.transpose(s_ref[0, 0], (1, 0))
    m = m_ref[0, 0]
    o_ref[0] = m * st + (1.0 - m) * q_ref[0]


def _run_blend(sampled_nd, q, mask2d, tnb):
    bsz, d, h, w = q.shape
    hw = h * w
    g = hw // tnb
    qr = q.reshape(bsz, d, hw)
    sr = sampled_nd.reshape(bsz, g, tnb, d)
    mr = mask2d.reshape(bsz, g, 1, tnb)
    out = pl.pallas_call(
        _blend_kernel,
        grid=(bsz, g),
        in_specs=[
            pl.BlockSpec((1, 1, tnb, d), lambda b, j: (b, j, 0, 0)),
            pl.BlockSpec((1, d, tnb), lambda b, j: (b, 0, j)),
            pl.BlockSpec((1, 1, 1, tnb), lambda b, j: (b, j, 0, 0)),
        ],
        out_specs=pl.BlockSpec((1, d, tnb), lambda b, j: (b, 0, j)),
        out_shape=jax.ShapeDtypeStruct((bsz, d, hw), jnp.float32),
        compiler_params=pltpu.CompilerParams(
            dimension_semantics=("parallel", "arbitrary")),
        interpret=_INTERPRET,
    )(sr, qr, mr)
    return out.reshape(bsz, d, h, w)


def _level_prep(q, cbk, strength, u):
    bsz, d, h, w = q.shape
    hw = h * w
    k = cbk.shape[0]
    flat = jnp.transpose(q, (0, 2, 3, 1)).reshape(bsz, hw, d)
    z2 = jnp.sum(flat ** 2, axis=-1)
    c2 = jnp.sum(cbk ** 2, axis=-1)
    skip = int(0.05 * k)
    s = strength.reshape(())
    n = jnp.maximum(jnp.floor(s * (k - skip)).astype(jnp.int32), 1)
    offs = jnp.floor(u * n.astype(jnp.float32)).astype(jnp.int32)
    pos = jnp.minimum(skip + offs, k - 1)
    return flat, z2, c2, pos


def _level(q, cbk, strength, u, mask2d):
    flat, z2, c2, pos = _level_prep(q, cbk, strength, u)
    idx = _run_select(q, cbk, c2, z2, pos, min(2048, q.shape[2] * q.shape[3]))
    sampled_nd = _sc_gather(cbk, idx)
    return _run_blend(sampled_nd, q, mask2d, min(4096, q.shape[2] * q.shape[3]))


def kernel(q_fine, q_coarse, M, cb_fine, cb_coarse, strength_fine, strength_coarse):
    M_float = M.astype(jnp.float32)
    fh, fw = q_fine.shape[-2], q_fine.shape[-1]
    ch, cw = q_coarse.shape[-2], q_coarse.shape[-1]
    mh, mw = M.shape[-2], M.shape[-1]
    m_fine = M_float[:, 0, :: mh // fh, :: mw // fw].reshape(M.shape[0], fh * fw)
    m_coarse = M_float[:, 0, :: mh // ch, :: mw // cw].reshape(M.shape[0], ch * cw)

    key = jax.random.key(42)
    kc, kf = jax.random.split(key)
    bsz = q_fine.shape[0]
    u_c = jax.random.uniform(kc, (bsz, ch * cw))
    u_f = jax.random.uniform(kf, (bsz, fh * fw))

    out_c = _level(q_coarse, cb_coarse, strength_coarse, u_c, m_coarse)
    out_f = _level(q_fine, cb_fine, strength_fine, u_f, m_fine)
    return (out_f, out_c)
